# baseline (speedup 1.0000x reference)
import functools

import jax
import jax.numpy as jnp
from jax import lax
from jax.experimental import pallas as pl
from jax.experimental.pallas import tpu as pltpu
from jax.experimental.pallas import tpu_sc as plsc

_V = 1000000
_B = 16384
_E = 16
_NC = 2
_NS = 16
_NW = _NC * _NS
_CHUNK = 128
_NCHUNK = 2
_BPW = _NCHUNK * _CHUNK
_BH = _B // 2
_BB = 16
_NBATCH = _BPW // _BB

_LT = _V // 128
_TAILBASE = _LT * 128
_TAIL = _V - _TAILBASE

_MLP_BLK = 8192


def _gather_body(idx_hbm, tt_hbm, tail_hbm, out_hbm,
                 idx_v, col_v, vflat_v, cols_v, res_v, tail_v,
                 sem0, sem1):
    wid = lax.axis_index("s") * _NC + lax.axis_index("c")
    pltpu.sync_copy(idx_hbm.at[wid], idx_v)
    pltpu.sync_copy(tail_hbm, tail_v)

    for k in range(_NCHUNK):
        for g in range(_CHUNK // 16):
            v = idx_v[k, pl.ds(g * 16, 16)]
            col_v[pl.ds(k * _CHUNK + g * 16, 16)] = lax.bitwise_and(v, 127)
            vflat_v[pl.ds(k * _CHUNK + g * 16, 16)] = v

    sems = (sem0, sem1)

    def enqueue(batch):
        buf = batch % 2
        copies = []
        m0 = batch * _BB
        v16 = idx_v[m0 // _CHUNK, pl.ds(m0 % _CHUNK, _BB)]
        j16 = jnp.minimum(lax.shift_right_logical(v16, 7), _LT - 1)
        for m in range(_BB):
            j = j16[m]
            off = pl.multiple_of(j * 128, 128)
            copies.append(
                pltpu.async_copy(
                    tt_hbm.at[:, pl.ds(off, 128)],
                    cols_v.at[buf, m],
                    sems[buf],
                )
            )
        return copies

    def extract(batch):
        buf = batch % 2
        m0 = batch * _BB
        rid = jax.lax.iota(jnp.int32, 16)
        cid = col_v[pl.ds(m0, 16)]
        v = vflat_v[pl.ds(m0, 16)]
        tl = jnp.clip(v - _TAILBASE, 0, _TAIL - 1)
        is_tail = v >= _TAILBASE
        for d in range(_E):
            dvec = jnp.full((16,), d, jnp.int32)
            val = plsc.load_gather(cols_v.at[buf], [rid, dvec, cid])
            tval = plsc.load_gather(tail_v, [tl, dvec])
            res_v[d // 8, m0 // 128, d % 8, pl.ds(m0 % 128, 16)] = (
                jnp.where(is_tail, tval, val)
            )

    inflight = enqueue(0)
    for batch in range(_NBATCH):
        nxt = enqueue(batch + 1) if batch + 1 < _NBATCH else []
        for c in inflight:
            c.wait()
        extract(batch)
        inflight = nxt

    for i in range(2):
        pltpu.sync_copy(
            res_v.at[i], out_hbm.at[pl.ds(i * 128 + wid * (_BPW // 128), _BPW // 128)]
        )


_gather = functools.partial(
    pl.kernel,
    out_type=jax.ShapeDtypeStruct((2 * (_BH // 128), 8, 128), jnp.float32),
    mesh=plsc.VectorSubcoreMesh(core_axis_name="c", subcore_axis_name="s"),
    scratch_types=[
        pltpu.VMEM((_NCHUNK, _CHUNK), jnp.int32),
        pltpu.VMEM((_BPW,), jnp.int32),
        pltpu.VMEM((_BPW,), jnp.int32),
        pltpu.VMEM((2, _BB, _E, 128), jnp.float32),
        pltpu.VMEM((2, _BPW // 128, 8, 128), jnp.float32),
        pltpu.VMEM((_TAIL, _E), jnp.float32),
        pltpu.SemaphoreType.DMA,
        pltpu.SemaphoreType.DMA,
    ],
    compiler_params=pltpu.CompilerParams(
        use_tc_tiling_on_sc=True, needs_layout_passes=False
    ),
)(_gather_body)


def _mlp_body(h_ref, w1_ref, b1_ref, w2_ref, b2_ref, w3_ref, b3_ref, o_ref):
    h = h_ref[...]
    h1 = jnp.maximum(
        jnp.dot(w1_ref[...], h, preferred_element_type=jnp.float32) + b1_ref[...],
        0.0,
    )
    h2 = jnp.maximum(
        jnp.dot(w2_ref[...], h1, preferred_element_type=jnp.float32) + b2_ref[...],
        0.0,
    )
    o = jnp.dot(w3_ref[...], h2, preferred_element_type=jnp.float32) + b3_ref[...]
    o_ref[...] = jax.nn.sigmoid(o)


def _mlp(h_t, w1, b1, w2, b2, w3, b3):
    grid = (_B // _MLP_BLK,)
    fixed = lambda shape: pl.BlockSpec(shape, lambda i: (0, 0))
    return pl.pallas_call(
        _mlp_body,
        grid=grid,
        in_specs=[
            pl.BlockSpec((_E, _MLP_BLK), lambda i: (0, i)),
            fixed((32, 16)),
            fixed((32, 1)),
            fixed((16, 32)),
            fixed((16, 1)),
            fixed((1, 16)),
            fixed((1, 1)),
        ],
        out_specs=pl.BlockSpec((1, _MLP_BLK), lambda i: (0, i)),
        out_shape=jax.ShapeDtypeStruct((1, _BH), jnp.float32),
    )(h_t, w1, b1, w2, b2, w3, b3)


def kernel(x_word, table, W1, b1, W2, b2, W3, b3):
    idx = x_word.astype(jnp.int32).reshape(_NW, 4, _CHUNK)
    tail = table[_TAILBASE:, :]
    tt = table.T
    w1, bv1 = W1, b1.reshape(32, 1)
    w2, bv2 = W2, b2.reshape(16, 1)
    w3, bv3 = W3, b3.reshape(1, 1)
    outs = []
    for half in range(2):
        ih = idx[:, half * 2:half * 2 + 2, :]
        h3 = _gather(ih, tt, tail)
        h_t = (
            h3.reshape(2, _BH // 128, 8, 128)
            .transpose(0, 2, 1, 3)
            .reshape(_E, _BH)
        )
        outs.append(_mlp(h_t, w1, bv1, w2, bv2, w3, bv3))
    out = jnp.concatenate(outs, axis=1)
    return out.reshape(_B)


# speedup vs baseline: 1.1601x; 1.1601x over previous
import functools

import jax
import jax.numpy as jnp
from jax import lax
from jax.experimental import pallas as pl
from jax.experimental.pallas import tpu as pltpu
from jax.experimental.pallas import tpu_sc as plsc

_V = 1000000
_B = 16384
_E = 16
_NC = 2
_NS = 16
_NW = _NC * _NS
_CHUNK = 128
_NCHUNK = _B // (_NW * _CHUNK)
_BPW = _NCHUNK * _CHUNK
_BB = 16
_NBATCH = _BPW // _BB

_LT = _V // 128
_TAILBASE = _LT * 128
_TAIL = _V - _TAILBASE

_MLP_BLK = 16384


def _gather_body(idx_hbm, tt_hbm, tail_hbm, out_hbm,
                 idx_v, col_v, vflat_v, cols_v, res_v, tail_v,
                 sem0, sem1):
    wid = lax.axis_index("s") * _NC + lax.axis_index("c")
    pltpu.sync_copy(idx_hbm.at[wid], idx_v)
    pltpu.sync_copy(tail_hbm, tail_v)

    for k in range(_NCHUNK):
        for g in range(_CHUNK // 16):
            v = idx_v[k, pl.ds(g * 16, 16)]
            col_v[pl.ds(k * _CHUNK + g * 16, 16)] = lax.bitwise_and(v, 127)
            vflat_v[pl.ds(k * _CHUNK + g * 16, 16)] = v

    sems = (sem0, sem1)

    def enqueue(batch):
        buf = batch % 2
        copies = []
        m0 = batch * _BB
        v16 = idx_v[m0 // _CHUNK, pl.ds(m0 % _CHUNK, _BB)]
        j16 = jnp.minimum(lax.shift_right_logical(v16, 7), _LT - 1)
        for m in range(_BB):
            j = j16[m]
            off = pl.multiple_of(j * 128, 128)
            copies.append(
                pltpu.async_copy(
                    tt_hbm.at[:, pl.ds(off, 128)],
                    cols_v.at[buf, m],
                    sems[buf],
                )
            )
        return copies

    def extract(batch):
        buf = batch % 2
        m0 = batch * _BB
        rid = jax.lax.iota(jnp.int32, 16)
        cid = col_v[pl.ds(m0, 16)]
        v = vflat_v[pl.ds(m0, 16)]
        tl = jnp.clip(v - _TAILBASE, 0, _TAIL - 1)
        is_tail = v >= _TAILBASE
        for d in range(_E):
            dvec = jnp.full((16,), d, jnp.int32)
            val = plsc.load_gather(cols_v.at[buf], [rid, dvec, cid])
            tval = plsc.load_gather(tail_v, [tl, dvec])
            res_v[d // 8, m0 // 128, d % 8, pl.ds(m0 % 128, 16)] = (
                jnp.where(is_tail, tval, val)
            )

    inflight = enqueue(0)
    for batch in range(_NBATCH):
        nxt = enqueue(batch + 1) if batch + 1 < _NBATCH else []
        for c in inflight:
            c.wait()
        extract(batch)
        inflight = nxt

    for i in range(2):
        pltpu.sync_copy(
            res_v.at[i], out_hbm.at[pl.ds(i * 128 + wid * (_BPW // 128), _BPW // 128)]
        )


_gather = functools.partial(
    pl.kernel,
    out_type=jax.ShapeDtypeStruct((2 * (_B // 128), 8, 128), jnp.float32),
    mesh=plsc.VectorSubcoreMesh(core_axis_name="c", subcore_axis_name="s"),
    scratch_types=[
        pltpu.VMEM((_NCHUNK, _CHUNK), jnp.int32),
        pltpu.VMEM((_BPW,), jnp.int32),
        pltpu.VMEM((_BPW,), jnp.int32),
        pltpu.VMEM((2, _BB, _E, 128), jnp.float32),
        pltpu.VMEM((2, _BPW // 128, 8, 128), jnp.float32),
        pltpu.VMEM((_TAIL, _E), jnp.float32),
        pltpu.SemaphoreType.DMA,
        pltpu.SemaphoreType.DMA,
    ],
    compiler_params=pltpu.CompilerParams(
        use_tc_tiling_on_sc=True, needs_layout_passes=False
    ),
)(_gather_body)


def _mlp_body(h_ref, w1_ref, b1_ref, w2_ref, b2_ref, w3_ref, b3_ref, o_ref):
    h = h_ref[...]
    h1 = jnp.maximum(
        jnp.dot(w1_ref[...], h, preferred_element_type=jnp.float32) + b1_ref[...],
        0.0,
    )
    h2 = jnp.maximum(
        jnp.dot(w2_ref[...], h1, preferred_element_type=jnp.float32) + b2_ref[...],
        0.0,
    )
    o = jnp.dot(w3_ref[...], h2, preferred_element_type=jnp.float32) + b3_ref[...]
    o_ref[...] = jax.nn.sigmoid(o)


def _mlp(h_t, w1, b1, w2, b2, w3, b3):
    grid = (_B // _MLP_BLK,)
    fixed = lambda shape: pl.BlockSpec(shape, lambda i: (0, 0))
    return pl.pallas_call(
        _mlp_body,
        grid=grid,
        in_specs=[
            pl.BlockSpec((_E, _MLP_BLK), lambda i: (0, i)),
            fixed((32, 16)),
            fixed((32, 1)),
            fixed((16, 32)),
            fixed((16, 1)),
            fixed((1, 16)),
            fixed((1, 1)),
        ],
        out_specs=pl.BlockSpec((1, _MLP_BLK), lambda i: (0, i)),
        out_shape=jax.ShapeDtypeStruct((1, _B), jnp.float32),
    )(h_t, w1, b1, w2, b2, w3, b3)


def kernel(x_word, table, W1, b1, W2, b2, W3, b3):
    idx = x_word.astype(jnp.int32).reshape(_NW, _NCHUNK, _CHUNK)
    tail = table[_TAILBASE:, :]
    h3 = _gather(idx, table.T, tail)
    h_t = (
        h3.reshape(2, _B // 128, 8, 128)
        .transpose(0, 2, 1, 3)
        .reshape(_E, _B)
    )
    out = _mlp(
        h_t,
        W1,
        b1.reshape(32, 1),
        W2,
        b2.reshape(16, 1),
        W3,
        b3.reshape(1, 1),
    )
    return out.reshape(_B)


# speedup vs baseline: 1.2112x; 1.0440x over previous
import functools

import jax
import jax.numpy as jnp
from jax import lax
from jax.experimental import pallas as pl
from jax.experimental.pallas import tpu as pltpu
from jax.experimental.pallas import tpu_sc as plsc

_V = 1000000
_B = 16384
_E = 16
_NC = 2
_NS = 16
_NW = _NC * _NS
_CHUNK = 128
_NCHUNK = _B // (_NW * _CHUNK)
_BPW = _NCHUNK * _CHUNK
_BB = 16
_NBATCH = _BPW // _BB

_LT = _V // 128
_TAILBASE = _LT * 128
_TAIL = _V - _TAILBASE

_MLP_BLK = 16384


def _gather_body(idx_hbm, tt_hbm, tail_hbm, out_hbm,
                 idx_v, col_v, vflat_v, cols_v, res_v, tail_v,
                 sem0, sem1, sem2):
    wid = lax.axis_index("s") * _NC + lax.axis_index("c")
    pltpu.sync_copy(idx_hbm.at[wid], idx_v)
    pltpu.sync_copy(tail_hbm, tail_v)

    for k in range(_NCHUNK):
        for g in range(_CHUNK // 16):
            v = idx_v[k, pl.ds(g * 16, 16)]
            col_v[pl.ds(k * _CHUNK + g * 16, 16)] = lax.bitwise_and(v, 127)
            vflat_v[pl.ds(k * _CHUNK + g * 16, 16)] = v

    sems = (sem0, sem1, sem2)

    def enqueue(batch):
        buf = batch % 3
        copies = []
        m0 = batch * _BB
        v16 = idx_v[m0 // _CHUNK, pl.ds(m0 % _CHUNK, _BB)]
        j16 = jnp.minimum(lax.shift_right_logical(v16, 7), _LT - 1)
        for m in range(_BB):
            j = j16[m]
            off = pl.multiple_of(j * 128, 128)
            copies.append(
                pltpu.async_copy(
                    tt_hbm.at[:, pl.ds(off, 128)],
                    cols_v.at[buf, m],
                    sems[buf],
                )
            )
        return copies

    def extract(batch):
        buf = batch % 3
        m0 = batch * _BB
        rid = jax.lax.iota(jnp.int32, 16)
        cid = col_v[pl.ds(m0, 16)]
        v = vflat_v[pl.ds(m0, 16)]
        tl = jnp.clip(v - _TAILBASE, 0, _TAIL - 1)
        is_tail = v >= _TAILBASE
        for d in range(_E):
            dvec = jnp.full((16,), d, jnp.int32)
            val = plsc.load_gather(cols_v.at[buf], [rid, dvec, cid])
            tval = plsc.load_gather(tail_v, [tl, dvec])
            res_v[d // 8, m0 // 128, d % 8, pl.ds(m0 % 128, 16)] = (
                jnp.where(is_tail, tval, val)
            )

    pending = {0: enqueue(0), 1: enqueue(1)}
    for batch in range(_NBATCH):
        if batch + 2 < _NBATCH:
            pending[batch + 2] = enqueue(batch + 2)
        for c in pending.pop(batch):
            c.wait()
        extract(batch)

    for i in range(2):
        pltpu.sync_copy(
            res_v.at[i], out_hbm.at[pl.ds(i * 128 + wid * (_BPW // 128), _BPW // 128)]
        )


_gather = functools.partial(
    pl.kernel,
    out_type=jax.ShapeDtypeStruct((2 * (_B // 128), 8, 128), jnp.float32),
    mesh=plsc.VectorSubcoreMesh(core_axis_name="c", subcore_axis_name="s"),
    scratch_types=[
        pltpu.VMEM((_NCHUNK, _CHUNK), jnp.int32),
        pltpu.VMEM((_BPW,), jnp.int32),
        pltpu.VMEM((_BPW,), jnp.int32),
        pltpu.VMEM((3, _BB, _E, 128), jnp.float32),
        pltpu.VMEM((2, _BPW // 128, 8, 128), jnp.float32),
        pltpu.VMEM((_TAIL, _E), jnp.float32),
        pltpu.SemaphoreType.DMA,
        pltpu.SemaphoreType.DMA,
        pltpu.SemaphoreType.DMA,
    ],
    compiler_params=pltpu.CompilerParams(
        use_tc_tiling_on_sc=True, needs_layout_passes=False
    ),
)(_gather_body)


def _mlp_body(h_ref, w1_ref, b1_ref, w2_ref, b2_ref, w3_ref, b3_ref, o_ref):
    h = h_ref[...]
    h1 = jnp.maximum(
        jnp.dot(w1_ref[...], h, preferred_element_type=jnp.float32) + b1_ref[...],
        0.0,
    )
    h2 = jnp.maximum(
        jnp.dot(w2_ref[...], h1, preferred_element_type=jnp.float32) + b2_ref[...],
        0.0,
    )
    o = jnp.dot(w3_ref[...], h2, preferred_element_type=jnp.float32) + b3_ref[...]
    o_ref[...] = jax.nn.sigmoid(o)


def _mlp(h_t, w1, b1, w2, b2, w3, b3):
    grid = (_B // _MLP_BLK,)
    fixed = lambda shape: pl.BlockSpec(shape, lambda i: (0, 0))
    return pl.pallas_call(
        _mlp_body,
        grid=grid,
        in_specs=[
            pl.BlockSpec((_E, _MLP_BLK), lambda i: (0, i)),
            fixed((32, 16)),
            fixed((32, 1)),
            fixed((16, 32)),
            fixed((16, 1)),
            fixed((1, 16)),
            fixed((1, 1)),
        ],
        out_specs=pl.BlockSpec((1, _MLP_BLK), lambda i: (0, i)),
        out_shape=jax.ShapeDtypeStruct((1, _B), jnp.float32),
    )(h_t, w1, b1, w2, b2, w3, b3)


def kernel(x_word, table, W1, b1, W2, b2, W3, b3):
    idx = x_word.astype(jnp.int32).reshape(_NW, _NCHUNK, _CHUNK)
    tail = table[_TAILBASE:, :]
    h3 = _gather(idx, table.T, tail)
    h_t = (
        h3.reshape(2, _B // 128, 8, 128)
        .transpose(0, 2, 1, 3)
        .reshape(_E, _B)
    )
    out = _mlp(
        h_t,
        W1,
        b1.reshape(32, 1),
        W2,
        b2.reshape(16, 1),
        W3,
        b3.reshape(1, 1),
    )
    return out.reshape(_B)
